# Initial kernel scaffold; baseline (speedup 1.0000x reference)
import jax
import jax.numpy as jnp
from jax.experimental import pallas as pl


def kernel(embeddings, loss_signal, memory, importance):
    raise NotImplementedError("write your pallas kernel here")


# speedup vs baseline: 1.0196x; 1.0196x over previous
import jax
import jax.numpy as jnp
from jax.experimental import pallas as pl

CAPACITY = 100000
D_MODEL = 512
BATCH = 4096

BLOCK_ROWS = 2048
NB_EMB = BATCH // BLOCK_ROWS
GRID = (CAPACITY + BLOCK_ROWS - 1) // BLOCK_ROWS

IMP_PAD = GRID * BLOCK_ROWS
IMP_COLS = 128
IMP_ROWS = IMP_PAD // IMP_COLS
IMP_BLOCK_ROWS = BLOCK_ROWS // IMP_COLS
SIG_ROWS = BATCH // IMP_COLS


def _body(emb_ref, sig_ref, mem_ref, imp_ref, out_mem_ref, out_imp_ref):
    i = pl.program_id(0)

    @pl.when(i < NB_EMB)
    def _():
        out_mem_ref[...] = emb_ref[...]
        out_imp_ref[...] = sig_ref[...]

    @pl.when(i >= NB_EMB)
    def _():
        out_mem_ref[...] = mem_ref[...]
        out_imp_ref[...] = imp_ref[...]


def kernel(embeddings, loss_signal, memory, importance):
    sig2d = loss_signal.reshape(SIG_ROWS, IMP_COLS)
    imp2d = jnp.pad(importance, (0, IMP_PAD - CAPACITY)).reshape(IMP_ROWS, IMP_COLS)

    emb_last = NB_EMB - 1
    out_mem, out_imp2d = pl.pallas_call(
        _body,
        grid=(GRID,),
        in_specs=[
            pl.BlockSpec((BLOCK_ROWS, D_MODEL), lambda i: (jnp.minimum(i, emb_last), 0)),
            pl.BlockSpec((IMP_BLOCK_ROWS, IMP_COLS), lambda i: (jnp.minimum(i, emb_last), 0)),
            pl.BlockSpec((BLOCK_ROWS, D_MODEL), lambda i: (i, 0)),
            pl.BlockSpec((IMP_BLOCK_ROWS, IMP_COLS), lambda i: (i, 0)),
        ],
        out_specs=[
            pl.BlockSpec((BLOCK_ROWS, D_MODEL), lambda i: (i, 0)),
            pl.BlockSpec((IMP_BLOCK_ROWS, IMP_COLS), lambda i: (i, 0)),
        ],
        out_shape=[
            jax.ShapeDtypeStruct((CAPACITY, D_MODEL), jnp.float32),
            jax.ShapeDtypeStruct((IMP_ROWS, IMP_COLS), jnp.float32),
        ],
    )(embeddings, sig2d, memory, imp2d)

    out_imp = out_imp2d.reshape(IMP_PAD)[:CAPACITY]
    return out_mem, out_imp
